# baseline (speedup 1.0000x reference)
import functools

import jax
import jax.numpy as jnp
from jax import lax
from jax.experimental import pallas as pl
from jax.experimental.pallas import tpu as pltpu
from jax.experimental.pallas import tpu_sc as plsc

B, C, H, W = 4, 96, 224, 224
R = B * C
RB = 24
GRID = R // RB
CB_COUNT = C // RB
QCNT = C // 4 + 1
L = 16
NW = 32
SC_ROWS = R // NW


def _tc_body(x0_ref, x1_ref, b1v_ref, b1s_ref, o1_ref, m1_scr):
    i = pl.program_id(0)

    @pl.when(i < CB_COUNT)
    def _compute_masks():
        a1 = jnp.abs(b1v_ref[...])

        def step(j, c1):
            return c1 + (jnp.abs(b1s_ref[j, 0]) <= a1).astype(jnp.int32)

        c1 = jax.lax.fori_loop(0, C, step, jnp.zeros((RB, 1), jnp.int32))
        m1_scr[pl.ds(i * RB, RB), :] = c1

    cb = (i % CB_COUNT) * RB
    m1 = jnp.reshape(m1_scr[pl.ds(cb, RB), :] >= QCNT, (RB, 1, 1))
    o1_ref[...] = jnp.where(m1, x0_ref[...], x1_ref[...])


def _tc_out1(x0r, x1r, b1):
    return pl.pallas_call(
        _tc_body,
        grid=(GRID,),
        in_specs=[
            pl.BlockSpec((RB, H, W), lambda i: (i, 0, 0)),
            pl.BlockSpec((RB, H, W), lambda i: (i, 0, 0)),
            pl.BlockSpec((RB, 1), lambda i: (i % CB_COUNT, 0)),
            pl.BlockSpec(memory_space=pltpu.SMEM),
        ],
        out_specs=pl.BlockSpec((RB, H, W), lambda i: (i, 0, 0)),
        out_shape=jax.ShapeDtypeStruct((R, H, W), jnp.float32),
        scratch_shapes=[pltpu.VMEM((C, 1), jnp.int32)],
    )(x0r, x1r, b1, b1)


def _sc_body(x0_hbm, x1_hbm, b2_hbm, o_hbm, bn_v, cnt_v, buf0, buf1,
             sem_in, sem_out):
    pltpu.sync_copy(b2_hbm, bn_v.at[pl.ds(0, C)])
    for iv in range(C // L):
        sl = pl.ds(iv * L, L)
        bn_v[sl] = jnp.abs(bn_v[sl])

    def jstep(j, carry):
        sj = bn_v[pl.ds(j, L)][0]
        out = []
        for iv in range(C // L):
            av = bn_v[pl.ds(iv * L, L)]
            out.append(carry[iv] +
                       jnp.where(sj <= av, jnp.int32(1), jnp.int32(0)))
        return tuple(out)

    init = tuple(jnp.zeros((L,), jnp.int32) for _ in range(C // L))
    cnt = lax.fori_loop(0, C, jstep, init)
    for iv in range(C // L):
        cnt_v[pl.ds(iv * L, L)] = cnt[iv]

    wid = lax.axis_index("c") * 16 + lax.axis_index("s")
    base = wid * SC_ROWS

    def start_copy(i):
        r = base + i
        m = cnt_v[pl.ds(r % C, L)][0] >= QCNT

        @pl.when(m)
        def _():
            pltpu.make_async_copy(x1_hbm.at[r], o_hbm.at[r], sem_in).start()

        @pl.when(jnp.logical_not(m))
        def _():
            pltpu.make_async_copy(x0_hbm.at[r], o_hbm.at[r], sem_in).start()

    for i in range(SC_ROWS):
        start_copy(i)
    for i in range(SC_ROWS):
        pltpu.make_async_copy(x0_hbm.at[base + i], o_hbm.at[base + i],
                              sem_in).wait()


def _sc_out2(x0r, x1r, b2):
    mesh = plsc.VectorSubcoreMesh(core_axis_name="c", subcore_axis_name="s")
    f = pl.kernel(
        _sc_body,
        out_type=jax.ShapeDtypeStruct((R, H, W), jnp.float32),
        mesh=mesh,
        scratch_types=[
            pltpu.VMEM((C + L,), jnp.float32),
            pltpu.VMEM((C + L,), jnp.int32),
            pltpu.VMEM((H, W), jnp.float32),
            pltpu.VMEM((H, W), jnp.float32),
            pltpu.SemaphoreType.DMA,
            pltpu.SemaphoreType.DMA,
        ],
        compiler_params=pltpu.CompilerParams(use_tc_tiling_on_sc=True),
    )
    return f(x0r, x1r, b2)


def kernel(x0, x1, bn1_weight, bn2_weight, bn_threshold):
    del bn_threshold
    x0r = x0.reshape(R, H, W)
    x1r = x1.reshape(R, H, W)
    b1 = bn1_weight.reshape(C, 1)
    out1 = _tc_out1(x0r, x1r, b1)
    out2 = _sc_out2(x0r, x1r, bn2_weight)
    return (out1.reshape(B, C, H, W), out2.reshape(B, C, H, W))


# speedup vs baseline: 16.9231x; 16.9231x over previous
import functools

import jax
import jax.numpy as jnp
from jax import lax
from jax.experimental import pallas as pl
from jax.experimental.pallas import tpu as pltpu
from jax.experimental.pallas import tpu_sc as plsc

B, C, H, W = 4, 96, 224, 224
R = B * C
RB = 24
GRID = R // RB
CB_COUNT = C // RB
QCNT = C // 4 + 1
L = 16
NW = 32
SC_ROWS = R // NW


def _tc_body(x0_ref, x1_ref, b1v_ref, b1s_ref, o1_ref, m1_scr):
    i = pl.program_id(0)

    @pl.when(i < CB_COUNT)
    def _compute_masks():
        a1 = jnp.abs(b1v_ref[...])

        def step(j, c1):
            return c1 + (jnp.abs(b1s_ref[j, 0]) <= a1).astype(jnp.int32)

        c1 = jax.lax.fori_loop(0, C, step, jnp.zeros((RB, 1), jnp.int32))
        m1_scr[pl.ds(i * RB, RB), :] = c1

    cb = (i % CB_COUNT) * RB
    m1 = jnp.reshape(m1_scr[pl.ds(cb, RB), :] >= QCNT, (RB, 1, 1))
    o1_ref[...] = jnp.where(m1, x0_ref[...], x1_ref[...])


def _tc_out1(x0r, x1r, b1):
    return pl.pallas_call(
        _tc_body,
        grid=(GRID,),
        in_specs=[
            pl.BlockSpec((RB, H, W), lambda i: (i, 0, 0)),
            pl.BlockSpec((RB, H, W), lambda i: (i, 0, 0)),
            pl.BlockSpec((RB, 1), lambda i: (i % CB_COUNT, 0)),
            pl.BlockSpec(memory_space=pltpu.SMEM),
        ],
        out_specs=pl.BlockSpec((RB, H, W), lambda i: (i, 0, 0)),
        out_shape=jax.ShapeDtypeStruct((R, H, W), jnp.float32),
        scratch_shapes=[pltpu.VMEM((C, 1), jnp.int32)],
    )(x0r, x1r, b1, b1)


def _sc_body(x0_hbm, x1_hbm, b2_hbm, o_hbm, bn_v, cnt_v, buf0, buf1,
             sem_in, sem_out):
    pltpu.sync_copy(b2_hbm, bn_v.at[pl.ds(0, C)])
    for iv in range(C // L):
        sl = pl.ds(iv * L, L)
        bn_v[sl] = jnp.abs(bn_v[sl])

    def jstep(j, carry):
        sj = bn_v[pl.ds(j, L)][0]
        out = []
        for iv in range(C // L):
            av = bn_v[pl.ds(iv * L, L)]
            out.append(carry[iv] +
                       jnp.where(sj <= av, jnp.int32(1), jnp.int32(0)))
        return tuple(out)

    init = tuple(jnp.zeros((L,), jnp.int32) for _ in range(C // L))
    cnt = lax.fori_loop(0, C, jstep, init)
    for iv in range(C // L):
        cnt_v[pl.ds(iv * L, L)] = cnt[iv]

    wid = lax.axis_index("c") * 16 + lax.axis_index("s")
    base = wid * SC_ROWS
    bufs = (buf0, buf1)

    def start_read(i):
        r = base + i
        m = cnt_v[pl.ds(r % C, L)][0] >= QCNT

        @pl.when(m)
        def _():
            pltpu.make_async_copy(x1_hbm.at[r], bufs[i % 2], sem_in).start()

        @pl.when(jnp.logical_not(m))
        def _():
            pltpu.make_async_copy(x0_hbm.at[r], bufs[i % 2], sem_in).start()

    start_read(0)
    for i in range(SC_ROWS):
        pltpu.make_async_copy(x0_hbm.at[base + i], bufs[i % 2], sem_in).wait()
        if i >= 1:
            pltpu.make_async_copy(
                bufs[(i - 1) % 2], o_hbm.at[base + i - 1], sem_out).wait()
        if i + 1 < SC_ROWS:
            start_read(i + 1)
        pltpu.make_async_copy(bufs[i % 2], o_hbm.at[base + i], sem_out).start()
    pltpu.make_async_copy(
        bufs[(SC_ROWS - 1) % 2], o_hbm.at[base + SC_ROWS - 1], sem_out).wait()


def _sc_out2(x0r, x1r, b2):
    mesh = plsc.VectorSubcoreMesh(core_axis_name="c", subcore_axis_name="s")
    f = pl.kernel(
        _sc_body,
        out_type=jax.ShapeDtypeStruct((R, H, W), jnp.float32),
        mesh=mesh,
        scratch_types=[
            pltpu.VMEM((C + L,), jnp.float32),
            pltpu.VMEM((C + L,), jnp.int32),
            pltpu.VMEM((H, W), jnp.float32),
            pltpu.VMEM((H, W), jnp.float32),
            pltpu.SemaphoreType.DMA,
            pltpu.SemaphoreType.DMA,
        ],
        compiler_params=pltpu.CompilerParams(use_tc_tiling_on_sc=True),
    )
    return f(x0r, x1r, b2)


def kernel(x0, x1, bn1_weight, bn2_weight, bn_threshold):
    del bn_threshold
    x0r = x0.reshape(R, H, W)
    x1r = x1.reshape(R, H, W)
    b1 = bn1_weight.reshape(C, 1)
    out2 = _sc_out2(x0r, x1r, bn2_weight)
    out1 = _tc_out1(x0r, x1r, b1)
    return (out1.reshape(B, C, H, W), out2.reshape(B, C, H, W))


# speedup vs baseline: 20.3565x; 1.2029x over previous
import jax
import jax.numpy as jnp
from jax import lax
from jax.experimental import pallas as pl
from jax.experimental.pallas import tpu as pltpu
from jax.experimental.pallas import tpu_sc as plsc

B, C, H, W = 4, 96, 224, 224
R = B * C
RB = 24
GRID = R // RB
CB_COUNT = C // RB
QCNT = C // 4 + 1
L = 16


def _sc_body(b1_hbm, b2_hbm, cnt_hbm, bn_v, cnt_v, sem):
    wid = lax.axis_index("c") * 16 + lax.axis_index("s")

    @pl.when(wid == 0)
    def _():
        pltpu.sync_copy(b1_hbm, bn_v.at[0, pl.ds(0, C)])
        pltpu.sync_copy(b2_hbm, bn_v.at[1, pl.ds(0, C)])
        for k in range(2):
            for iv in range(C // L):
                sl = pl.ds(iv * L, L)
                bn_v[k, sl] = jnp.abs(bn_v[k, sl])

        def jstep(j, carry):
            out = []
            for k in range(2):
                sj = bn_v[k, pl.ds(j, L)][0]
                for iv in range(C // L):
                    av = bn_v[k, pl.ds(iv * L, L)]
                    out.append(carry[k * (C // L) + iv] +
                               jnp.where(sj <= av, jnp.int32(1),
                                         jnp.int32(0)))
            return tuple(out)

        init = tuple(jnp.zeros((L,), jnp.int32) for _ in range(2 * (C // L)))
        cnt = lax.fori_loop(0, C, jstep, init)
        for k in range(2):
            for iv in range(C // L):
                cnt_v[pl.ds(k * C + iv * L, L)] = cnt[k * (C // L) + iv]
        pltpu.sync_copy(cnt_v.at[pl.ds(0, 2 * C)], cnt_hbm)


def _sc_counts(b1, b2):
    mesh = plsc.VectorSubcoreMesh(core_axis_name="c", subcore_axis_name="s")
    f = pl.kernel(
        _sc_body,
        out_type=jax.ShapeDtypeStruct((2 * C,), jnp.int32),
        mesh=mesh,
        scratch_types=[
            pltpu.VMEM((2, C + L), jnp.float32),
            pltpu.VMEM((2 * C + L,), jnp.int32),
            pltpu.SemaphoreType.DMA,
        ],
        compiler_params=pltpu.CompilerParams(use_tc_tiling_on_sc=True),
    )
    return f(b1, b2)


def _tc_body(x0_ref, x1_ref, c1_ref, c2_ref, o1_ref, o2_ref):
    m1 = jnp.reshape(c1_ref[...] >= QCNT, (RB, 1, 1))
    m2 = jnp.reshape(c2_ref[...] >= QCNT, (RB, 1, 1))
    x0 = x0_ref[...]
    x1 = x1_ref[...]
    o1_ref[...] = jnp.where(m1, x0, x1)
    o2_ref[...] = jnp.where(m2, x1, x0)


def _tc_exchange(x0r, x1r, cnt2d):
    return pl.pallas_call(
        _tc_body,
        grid=(GRID,),
        in_specs=[
            pl.BlockSpec((RB, H, W), lambda i: (i, 0, 0)),
            pl.BlockSpec((RB, H, W), lambda i: (i, 0, 0)),
            pl.BlockSpec((RB, 1), lambda i: (i % CB_COUNT, 0)),
            pl.BlockSpec((RB, 1), lambda i: (CB_COUNT + i % CB_COUNT, 0)),
        ],
        out_specs=[
            pl.BlockSpec((RB, H, W), lambda i: (i, 0, 0)),
            pl.BlockSpec((RB, H, W), lambda i: (i, 0, 0)),
        ],
        out_shape=[
            jax.ShapeDtypeStruct((R, H, W), jnp.float32),
            jax.ShapeDtypeStruct((R, H, W), jnp.float32),
        ],
    )(x0r, x1r, cnt2d, cnt2d)


def kernel(x0, x1, bn1_weight, bn2_weight, bn_threshold):
    del bn_threshold
    x0r = x0.reshape(R, H, W)
    x1r = x1.reshape(R, H, W)
    cnt2d = _sc_counts(bn1_weight, bn2_weight).reshape(2 * C, 1)
    out1, out2 = _tc_exchange(x0r, x1r, cnt2d)
    return (out1.reshape(B, C, H, W), out2.reshape(B, C, H, W))
